# baseline (speedup 1.0000x reference)
import functools

import jax
import jax.numpy as jnp
from jax import lax
from jax.experimental import pallas as pl
from jax.experimental.pallas import tpu as pltpu
from jax.experimental.pallas import tpu_sc as plsc

N = 10000
D_IN = 128
H = 4
C = 128
HC = H * C
BG = 64
EMBED = 10

N_PAD = 10112
DL = 144
DR = 128
NTILES = 16
EB = 64
ROWS_PER_TILE = N_PAD // NTILES
NBLK_PREP = N_PAD // 128


def _prep_body(x_ref, wl_ref, wr_ref, tl_ref, tr_ref):
    xb = x_ref[...]
    yl = jnp.dot(xb, wl_ref[...], preferred_element_type=jnp.float32)
    yr = jnp.dot(xb, wr_ref[...], preferred_element_type=jnp.float32)
    ones_col = jnp.where(
        lax.broadcasted_iota(jnp.int32, (128, 16), 1) == 0, 1.0, 0.0
    ).astype(jnp.float32)
    for h in range(H):
        tl_ref[h, :, 0:128] = yl[:, h * 128:(h + 1) * 128]
        tl_ref[h, :, 128:144] = ones_col
        tr_ref[h, :, :] = yr[:, h * 128:(h + 1) * 128]


def _prep(x_pad, W_l, W_r):
    return pl.pallas_call(
        _prep_body,
        grid=(NBLK_PREP,),
        in_specs=[
            pl.BlockSpec((128, D_IN), lambda i: (i, 0)),
            pl.BlockSpec((D_IN, HC), lambda i: (0, 0)),
            pl.BlockSpec((D_IN, HC), lambda i: (0, 0)),
        ],
        out_specs=[
            pl.BlockSpec((H, 128, DL), lambda i: (0, i, 0)),
            pl.BlockSpec((H, 128, DR), lambda i: (0, i, 0)),
        ],
        out_shape=[
            jax.ShapeDtypeStruct((H, N_PAD, DL), jnp.float32),
            jax.ShapeDtypeStruct((H, N_PAD, DR), jnp.float32),
        ],
    )(x_pad, W_l, W_r)


def _sc_edge_kernel(ept, nb):
    mesh = plsc.VectorSubcoreMesh(
        core_axis_name="c", subcore_axis_name="s", num_cores=2,
        num_subcores=NTILES)

    @functools.partial(
        pl.kernel,
        mesh=mesh,
        compiler_params=pltpu.CompilerParams(
            needs_layout_passes=False, use_tc_tiling_on_sc=False),
        out_type=jax.ShapeDtypeStruct((H * N_PAD, DL), jnp.float32),
        scratch_types=[
            pltpu.VMEM_SHARED((N_PAD, DL), jnp.float32),
            pltpu.VMEM((EB, DL), jnp.float32),
            pltpu.VMEM((EB, DL), jnp.float32),
            pltpu.VMEM((EB, DR), jnp.float32),
            pltpu.VMEM((EB, DR), jnp.float32),
            pltpu.VMEM((EB,), jnp.int32),
            pltpu.VMEM((EB,), jnp.int32),
            pltpu.VMEM((EB,), jnp.int32),
            pltpu.VMEM((EB,), jnp.int32),
            pltpu.VMEM((EB,), jnp.int32),
            pltpu.VMEM((EB,), jnp.int32),
            pltpu.VMEM((8, 16), jnp.float32),
            pltpu.SemaphoreType.DMA,
            pltpu.SemaphoreType.DMA,
            pltpu.SemaphoreType.DMA,
            pltpu.SemaphoreType.DMA,
            pltpu.SemaphoreType.DMA,
            pltpu.SemaphoreType.DMA,
        ],
    )
    def sc_kernel(tl_hbm, tr_hbm, srcoff_hbm, dstoff_hbm, draw_hbm, att_hbm,
                  out_hbm,
                  accum, l0, l1, r0, r1,
                  so0, so1, do0, do1, db0, db1, attb,
                  sl0, sl1, sr0, sr1, si0, si1):
        cid = lax.axis_index("c")
        sid = lax.axis_index("s")
        lbuf = (l0, l1)
        rbuf = (r0, r1)
        sobuf = (so0, so1)
        dobuf = (do0, do1)
        dbbuf = (db0, db1)
        sem_l = (sl0, sl1)
        sem_r = (sr0, sr1)
        sem_i = (si0, si1)
        rbase = sid * ROWS_PER_TILE

        def issue_idx_gather(rc, blk, b):
            base = ((rc * NTILES + sid) * nb + blk) * EB
            pltpu.async_copy(srcoff_hbm.at[pl.ds(base, EB)], sobuf[b],
                             sem_i[b])
            pltpu.async_copy(dstoff_hbm.at[pl.ds(base, EB)], dobuf[b],
                             sem_i[b])

        def issue_idx_draw(blk, b):
            dbase = (sid * nb + blk) * EB
            pltpu.async_copy(draw_hbm.at[pl.ds(dbase, EB)], dbbuf[b],
                             sem_i[b])

        def issue_idx(rc, blk, b):
            issue_idx_gather(rc, blk, b)
            issue_idx_draw(blk, b)

        def wait_idx(b):
            pltpu.make_async_copy(srcoff_hbm.at[pl.ds(0, EB)], sobuf[b],
                                  sem_i[b]).wait()
            pltpu.make_async_copy(dstoff_hbm.at[pl.ds(0, EB)], dobuf[b],
                                  sem_i[b]).wait()
            pltpu.make_async_copy(draw_hbm.at[pl.ds(0, EB)], dbbuf[b],
                                  sem_i[b]).wait()

        def issue_gathers(b):
            pltpu.async_copy(tl_hbm.at[sobuf[b]], lbuf[b], sem_l[b])
            pltpu.async_copy(tr_hbm.at[dobuf[b]], rbuf[b], sem_r[b])

        def wait_gathers(b):
            pltpu.make_async_copy(tl_hbm.at[sobuf[b]], lbuf[b],
                                  sem_l[b]).wait()
            pltpu.make_async_copy(tr_hbm.at[dobuf[b]], rbuf[b],
                                  sem_r[b]).wait()

        def compute_block(b):
            avecs = [attb[k, :] for k in range(8)]

            def ebody(be, carry):
                lv = [lbuf[b][be, pl.ds(16 * k, 16)] for k in range(9)]
                acc = None
                for k in range(8):
                    t = lv[k] + rbuf[b][be, pl.ds(16 * k, 16)]
                    y = jnp.maximum(t, 0.2 * t)
                    p = y * avecs[k]
                    acc = p if acc is None else acc + p
                alpha = jnp.sum(acc)
                exv = jnp.exp(jnp.broadcast_to(alpha, (16,)))
                for k in range(9):
                    lbuf[b][be, pl.ds(16 * k, 16)] = lv[k] * exv
                return carry

            lax.fori_loop(0, EB, ebody, 0, unroll=4)

        zvec = jnp.zeros((16,), jnp.float32)

        for rnd in range(2):
            h = 2 * rnd + cid
            rc = 2 * rnd + cid

            def zbody(i, carry):
                for k in range(9):
                    l0[i, pl.ds(16 * k, 16)] = zvec
                return carry
            lax.fori_loop(0, EB, zbody, 0)
            for k in range(ROWS_PER_TILE // EB):
                pltpu.sync_copy(l0.at[pl.ds(0, EB)],
                                accum.at[pl.ds(rbase + k * EB, EB)])
            rem = ROWS_PER_TILE % EB
            if rem:
                pltpu.sync_copy(
                    l0.at[pl.ds(0, rem)],
                    accum.at[pl.ds(rbase + (ROWS_PER_TILE // EB) * EB, rem)])
            pltpu.sync_copy(att_hbm.at[pl.ds(h * 8, 8)], attb)
            plsc.subcore_barrier()

            issue_idx(rc, 0, 0)
            issue_idx(rc, 1, 1)
            wait_idx(0)
            issue_gathers(0)

            def gbody(g, carry):
                not_last = g < nb // 2 - 1
                for b in range(2):
                    wait_gathers(b)

                    def next_stage(b=b):
                        wait_idx(1 - b)
                        issue_gathers(1 - b)

                    if b == 0:
                        next_stage()
                    else:
                        pl.when(not_last)(next_stage)
                    pl.when(not_last)(
                        lambda b=b, g=g: issue_idx_gather(rc, 2 * g + b + 2, b))
                    compute_block(b)
                    pltpu.sync_copy(lbuf[b], accum.at[dbbuf[b]], add=True)
                    pl.when(not_last)(
                        lambda b=b, g=g: issue_idx_draw(2 * g + b + 2, b))
                return carry

            lax.fori_loop(0, nb // 2, gbody, 0)
            plsc.subcore_barrier()
            pltpu.sync_copy(
                accum.at[pl.ds(rbase, ROWS_PER_TILE)],
                out_hbm.at[pl.ds(h * N_PAD + rbase, ROWS_PER_TILE)])

    return sc_kernel


POST_R = 1264
POST_G = N_PAD // POST_R


def _post_body(acc_ref, batch_ref, bias_ref, w1_ref, b1_ref, w2_ref, b2_ref,
               out_ref, pooled_s, cnt_s):
    i = pl.program_id(0)

    @pl.when(i == 0)
    def _():
        pooled_s[...] = jnp.zeros_like(pooled_s)
        cnt_s[...] = jnp.zeros_like(cnt_s)

    b = batch_ref[...]
    gid = lax.broadcasted_iota(jnp.int32, (1, BG), 1).astype(jnp.float32)
    P = (b == gid).astype(jnp.float32)
    parts = []
    for h in range(H):
        den = acc_ref[h, :, 128:129]
        parts.append(acc_ref[h, :, 0:128] / jnp.maximum(den, 1e-30))
    nodes = jnp.concatenate(parts, axis=1)
    pooled_s[...] += lax.dot_general(
        P, nodes, (((0,), (0,)), ((), ())), preferred_element_type=jnp.float32)
    cnt_s[...] += jnp.sum(P, axis=0, keepdims=True)

    @pl.when(i == POST_G - 1)
    def _():
        cnt = jnp.maximum(cnt_s[...], 1.0)
        pm = pooled_s[...] / cnt.reshape(BG, 1) + bias_ref[...]
        hmid = jnp.maximum(
            jnp.dot(pm, w1_ref[...], preferred_element_type=jnp.float32)
            + b1_ref[...], 0.0)
        out_ref[...] = (
            jnp.dot(hmid, w2_ref[...], preferred_element_type=jnp.float32)
            + b2_ref[...])


def _post(acc, batch_f, bias, mlp_W1, mlp_b1, mlp_W2, mlp_b2):
    return pl.pallas_call(
        _post_body,
        grid=(POST_G,),
        in_specs=[
            pl.BlockSpec((H, POST_R, DL), lambda i: (0, i, 0)),
            pl.BlockSpec((POST_R, 1), lambda i: (i, 0)),
            pl.BlockSpec((1, HC), lambda i: (0, 0)),
            pl.BlockSpec((HC, C), lambda i: (0, 0)),
            pl.BlockSpec((1, C), lambda i: (0, 0)),
            pl.BlockSpec((C, EMBED), lambda i: (0, 0)),
            pl.BlockSpec((1, EMBED), lambda i: (0, 0)),
        ],
        out_specs=pl.BlockSpec((BG, EMBED), lambda i: (0, 0)),
        out_shape=jax.ShapeDtypeStruct((BG, EMBED), jnp.float32),
        scratch_shapes=[
            pltpu.VMEM((BG, HC), jnp.float32),
            pltpu.VMEM((1, BG), jnp.float32),
        ],
    )(acc, batch_f, bias, mlp_W1, mlp_b1, mlp_W2, mlp_b2)


def kernel(x, edge_index, batch, W_l, W_r, att, bias,
           mlp_W1, mlp_b1, mlp_W2, mlp_b2):
    E = edge_index.shape[1]
    Et = E + N
    ept = -(-Et // (NTILES * 2 * EB)) * 2 * EB
    Et_pad = ept * NTILES
    nb = ept // EB

    loop = jnp.arange(N, dtype=jnp.int32)
    pad = jnp.full((Et_pad - Et,), N, dtype=jnp.int32)
    src = jnp.concatenate([edge_index[0].astype(jnp.int32), loop, pad])
    dst = jnp.concatenate([edge_index[1].astype(jnp.int32), loop, pad])

    hoffs = (jnp.arange(4, dtype=jnp.int32) * N_PAD)[:, None]
    srcoff = (hoffs + src[None, :]).reshape(-1)
    dstoff = (hoffs + dst[None, :]).reshape(-1)

    x_pad = jnp.zeros((N_PAD, D_IN), jnp.float32).at[:N].set(x)
    TL, TR = _prep(x_pad, W_l, W_r)
    TL2 = TL.reshape(H * N_PAD, DL)
    TR2 = TR.reshape(H * N_PAD, DR)
    att_r = att.reshape(H * 8, 16)

    acc = _sc_edge_kernel(ept, nb)(TL2, TR2, srcoff, dstoff, dst, att_r)
    acc = acc.reshape(H, N_PAD, DL)

    batch_f = jnp.full((N_PAD, 1), -1.0, jnp.float32).at[:N, 0].set(
        batch.astype(jnp.float32))
    return _post(acc, batch_f, bias.reshape(1, HC), mlp_W1,
                 mlp_b1.reshape(1, C), mlp_W2, mlp_b2.reshape(1, EMBED))


# speedup vs baseline: 1.0572x; 1.0572x over previous
import functools

import jax
import jax.numpy as jnp
from jax import lax
from jax.experimental import pallas as pl
from jax.experimental.pallas import tpu as pltpu
from jax.experimental.pallas import tpu_sc as plsc

N = 10000
D_IN = 128
H = 4
C = 128
HC = H * C
BG = 64
EMBED = 10

N_PAD = 10112
DLF = 160
DLW = DLF // 2
DRW = C // 2
DM = 144
NTILES = 16
EB = 64
ROWS_PER_TILE = N_PAD // NTILES
NBLK_PREP = N_PAD // 128


def _prep_body(x_ref, wl_ref, wr_ref, tl_ref, tr_ref):
    xb = x_ref[...]
    yl = jnp.dot(xb, wl_ref[...], preferred_element_type=jnp.float32)
    yr = jnp.dot(xb, wr_ref[...], preferred_element_type=jnp.float32)
    ones_col = jnp.where(
        lax.broadcasted_iota(jnp.int32, (128, 32), 1) == 0, 1.0, 0.0
    ).astype(jnp.float32)
    for h in range(H):
        tl_ref[h, :, 0:128] = yl[:, h * 128:(h + 1) * 128]
        tl_ref[h, :, 128:160] = ones_col
        tr_ref[h, :, :] = yr[:, h * 128:(h + 1) * 128]


def _prep(x_pad, W_l, W_r):
    return pl.pallas_call(
        _prep_body,
        grid=(NBLK_PREP,),
        in_specs=[
            pl.BlockSpec((128, D_IN), lambda i: (i, 0)),
            pl.BlockSpec((D_IN, HC), lambda i: (0, 0)),
            pl.BlockSpec((D_IN, HC), lambda i: (0, 0)),
        ],
        out_specs=[
            pl.BlockSpec((H, 128, DLF), lambda i: (0, i, 0)),
            pl.BlockSpec((H, 128, C), lambda i: (0, i, 0)),
        ],
        out_shape=[
            jax.ShapeDtypeStruct((H, N_PAD, DLF), jnp.float32),
            jax.ShapeDtypeStruct((H, N_PAD, C), jnp.float32),
        ],
    )(x_pad, W_l, W_r)


def _pack_bf16(arr2d):
    b = arr2d.astype(jnp.bfloat16)
    return lax.bitcast_convert_type(
        b.reshape(arr2d.shape[0], arr2d.shape[1] // 2, 2), jnp.int32)


def _sc_edge_kernel(ept, nb):
    mesh = plsc.VectorSubcoreMesh(
        core_axis_name="c", subcore_axis_name="s", num_cores=2,
        num_subcores=NTILES)

    @functools.partial(
        pl.kernel,
        mesh=mesh,
        compiler_params=pltpu.CompilerParams(
            needs_layout_passes=False, use_tc_tiling_on_sc=False),
        out_type=jax.ShapeDtypeStruct((H * N_PAD, DM), jnp.float32),
        scratch_types=[
            pltpu.VMEM_SHARED((N_PAD, DM), jnp.float32),
            pltpu.VMEM((EB, DLW), jnp.int32),
            pltpu.VMEM((EB, DLW), jnp.int32),
            pltpu.VMEM((EB, DRW), jnp.int32),
            pltpu.VMEM((EB, DRW), jnp.int32),
            pltpu.VMEM((EB, DM), jnp.float32),
            pltpu.VMEM((EB, DM), jnp.float32),
            pltpu.VMEM((EB,), jnp.int32),
            pltpu.VMEM((EB,), jnp.int32),
            pltpu.VMEM((EB,), jnp.int32),
            pltpu.VMEM((EB,), jnp.int32),
            pltpu.VMEM((EB,), jnp.int32),
            pltpu.VMEM((EB,), jnp.int32),
            pltpu.VMEM((EB,), jnp.int32),
            pltpu.VMEM((EB,), jnp.int32),
            pltpu.VMEM((8, 16), jnp.float32),
            pltpu.SemaphoreType.DMA,
            pltpu.SemaphoreType.DMA,
            pltpu.SemaphoreType.DMA,
            pltpu.SemaphoreType.DMA,
            pltpu.SemaphoreType.DMA,
            pltpu.SemaphoreType.DMA,
            pltpu.SemaphoreType.DMA,
            pltpu.SemaphoreType.DMA,
            pltpu.SemaphoreType.DMA,
            pltpu.SemaphoreType.DMA,
            pltpu.SemaphoreType.DMA,
            pltpu.SemaphoreType.DMA,
        ],
    )
    def sc_kernel(tl_hbm, tr_hbm, srcoff_hbm, dstoff_hbm, draw_hbm, att_hbm,
                  out_hbm,
                  accum, l0, l1, r0, r1, m0, m1,
                  so0, so1, do0, do1, db0, db1, db2, db3, attb,
                  sl0, sl1, sr0, sr1, si0, si1, ss0, ss1,
                  sd0, sd1, sd2, sd3):
        cid = lax.axis_index("c")
        sid = lax.axis_index("s")
        lbuf = (l0, l1)
        rbuf = (r0, r1)
        mbuf = (m0, m1)
        sobuf = (so0, so1)
        dobuf = (do0, do1)
        dbuf = (db0, db1, db2, db3)
        sem_l = (sl0, sl1)
        sem_r = (sr0, sr1)
        sem_i = (si0, si1)
        sem_s = (ss0, ss1)
        sem_d = (sd0, sd1, sd2, sd3)
        rbase = sid * ROWS_PER_TILE

        def issue_scatter(b, q):
            pltpu.async_copy(mbuf[b], accum.at[dbuf[q]], sem_s[b], add=True)

        def wait_scatter(b, q):
            pltpu.make_async_copy(mbuf[b], accum.at[dbuf[q]],
                                  sem_s[b]).wait()

        def issue_idx_gather(rc, blk, b):
            base = ((rc * NTILES + sid) * nb + blk) * EB
            pltpu.async_copy(srcoff_hbm.at[pl.ds(base, EB)], sobuf[b],
                             sem_i[b])
            pltpu.async_copy(dstoff_hbm.at[pl.ds(base, EB)], dobuf[b],
                             sem_i[b])

        def issue_idx_draw(blk, q):
            dbase = (sid * nb + blk) * EB
            pltpu.async_copy(draw_hbm.at[pl.ds(dbase, EB)], dbuf[q],
                             sem_d[q])

        def wait_idx(b):
            pltpu.make_async_copy(srcoff_hbm.at[pl.ds(0, EB)], sobuf[b],
                                  sem_i[b]).wait()
            pltpu.make_async_copy(dstoff_hbm.at[pl.ds(0, EB)], dobuf[b],
                                  sem_i[b]).wait()

        def wait_draw(q):
            pltpu.make_async_copy(draw_hbm.at[pl.ds(0, EB)], dbuf[q],
                                  sem_d[q]).wait()

        def issue_gathers(b):
            pltpu.async_copy(tl_hbm.at[sobuf[b]], lbuf[b], sem_l[b])
            pltpu.async_copy(tr_hbm.at[dobuf[b]], rbuf[b], sem_r[b])

        def wait_gathers(b):
            pltpu.make_async_copy(tl_hbm.at[sobuf[b]], lbuf[b],
                                  sem_l[b]).wait()
            pltpu.make_async_copy(tr_hbm.at[dobuf[b]], rbuf[b],
                                  sem_r[b]).wait()

        himask = jnp.int32(-65536)

        def unpack2(w):
            lo = plsc.bitcast(jnp.left_shift(w, 16), jnp.float32)
            hi = plsc.bitcast(jnp.bitwise_and(w, himask), jnp.float32)
            return lo, hi

        def compute_block(b):
            avecs = [attb[k, :] for k in range(8)]

            def ebody(be, carry):
                lw = [lbuf[b][be, pl.ds(16 * k, 16)] for k in range(5)]
                rw = [rbuf[b][be, pl.ds(16 * k, 16)] for k in range(4)]
                fl = []
                for k in range(4):
                    lo, hi = unpack2(lw[k])
                    fl += [lo, hi]
                fr = []
                for k in range(4):
                    lo, hi = unpack2(rw[k])
                    fr += [lo, hi]
                acc = None
                for j in range(8):
                    t = fl[j] + fr[j]
                    y = jnp.maximum(t, 0.2 * t)
                    p = y * avecs[j]
                    acc = p if acc is None else acc + p
                alpha = jnp.sum(acc)
                exv = jnp.exp(jnp.broadcast_to(alpha, (16,)))
                for j in range(8):
                    mbuf[b][be, pl.ds(16 * j, 16)] = fl[j] * exv
                dlo, dhi = unpack2(lw[4])
                mbuf[b][be, pl.ds(128, 16)] = (dlo + dhi) * exv
                return carry

            lax.fori_loop(0, EB, ebody, 0, unroll=4)

        zvec = jnp.zeros((16,), jnp.float32)

        for rnd in range(2):
            h = 2 * rnd + cid
            rc = 2 * rnd + cid

            def zbody(i, carry):
                for k in range(DM // 16):
                    m0[i, pl.ds(16 * k, 16)] = zvec
                return carry
            lax.fori_loop(0, EB, zbody, 0)
            for k in range(ROWS_PER_TILE // EB):
                pltpu.sync_copy(m0.at[pl.ds(0, EB)],
                                accum.at[pl.ds(rbase + k * EB, EB)])
            rem = ROWS_PER_TILE % EB
            if rem:
                pltpu.sync_copy(
                    m0.at[pl.ds(0, rem)],
                    accum.at[pl.ds(rbase + (ROWS_PER_TILE // EB) * EB, rem)])
            pltpu.sync_copy(att_hbm.at[pl.ds(h * 8, 8)], attb)
            plsc.subcore_barrier()

            issue_idx_gather(rc, 0, 0)
            issue_idx_gather(rc, 1, 1)
            issue_idx_draw(0, 0)
            issue_idx_draw(1, 1)
            wait_idx(0)
            issue_gathers(0)

            def gbody(g, carry):
                not_last = g < nb // 4 - 1
                for u in range(4):
                    b = u & 1
                    q2 = (u + 2) % 4
                    wait_gathers(b)

                    def next_gathers(g=g, b=b, u=u):
                        wait_idx(1 - b)
                        issue_gathers(1 - b)

                    if u < 3:
                        next_gathers()
                    else:
                        pl.when(not_last)(next_gathers)

                    def prefetch(g=g, b=b, u=u, q2=q2):
                        issue_idx_gather(rc, 4 * g + u + 2, b)
                        issue_idx_draw(4 * g + u + 2, q2)

                    if u < 2:
                        pl.when(g >= 1)(lambda b=b, q2=q2: wait_scatter(b, q2))
                        prefetch()
                    else:
                        wait_scatter(b, q2)
                        pl.when(not_last)(prefetch)
                    compute_block(b)
                    wait_draw(u)
                    issue_scatter(b, u)
                return carry

            lax.fori_loop(0, nb // 4, gbody, 0)
            wait_scatter(0, (nb - 2) % 4)
            wait_scatter(1, (nb - 1) % 4)
            plsc.subcore_barrier()
            pltpu.sync_copy(
                accum.at[pl.ds(rbase, ROWS_PER_TILE)],
                out_hbm.at[pl.ds(h * N_PAD + rbase, ROWS_PER_TILE)])

    return sc_kernel


POST_R = 1264
POST_G = N_PAD // POST_R


def _post_body(acc_ref, batch_ref, bias_ref, w1_ref, b1_ref, w2_ref, b2_ref,
               out_ref, pooled_s, cnt_s):
    i = pl.program_id(0)

    @pl.when(i == 0)
    def _():
        pooled_s[...] = jnp.zeros_like(pooled_s)
        cnt_s[...] = jnp.zeros_like(cnt_s)

    b = batch_ref[...]
    gid = lax.broadcasted_iota(jnp.int32, (1, BG), 1).astype(jnp.float32)
    P = (b == gid).astype(jnp.float32)
    parts = []
    for h in range(H):
        den = jnp.sum(acc_ref[h, :, 128:144], axis=1, keepdims=True)
        parts.append(acc_ref[h, :, 0:128] / jnp.maximum(den, 1e-30))
    nodes = jnp.concatenate(parts, axis=1)
    pooled_s[...] += lax.dot_general(
        P, nodes, (((0,), (0,)), ((), ())), preferred_element_type=jnp.float32)
    cnt_s[...] += jnp.sum(P, axis=0, keepdims=True)

    @pl.when(i == POST_G - 1)
    def _():
        cnt = jnp.maximum(cnt_s[...], 1.0)
        pm = pooled_s[...] / cnt.reshape(BG, 1) + bias_ref[...]
        hmid = jnp.maximum(
            jnp.dot(pm, w1_ref[...], preferred_element_type=jnp.float32)
            + b1_ref[...], 0.0)
        out_ref[...] = (
            jnp.dot(hmid, w2_ref[...], preferred_element_type=jnp.float32)
            + b2_ref[...])


def _post(acc, batch_f, bias, mlp_W1, mlp_b1, mlp_W2, mlp_b2):
    return pl.pallas_call(
        _post_body,
        grid=(POST_G,),
        in_specs=[
            pl.BlockSpec((H, POST_R, DM), lambda i: (0, i, 0)),
            pl.BlockSpec((POST_R, 1), lambda i: (i, 0)),
            pl.BlockSpec((1, HC), lambda i: (0, 0)),
            pl.BlockSpec((HC, C), lambda i: (0, 0)),
            pl.BlockSpec((1, C), lambda i: (0, 0)),
            pl.BlockSpec((C, EMBED), lambda i: (0, 0)),
            pl.BlockSpec((1, EMBED), lambda i: (0, 0)),
        ],
        out_specs=pl.BlockSpec((BG, EMBED), lambda i: (0, 0)),
        out_shape=jax.ShapeDtypeStruct((BG, EMBED), jnp.float32),
        scratch_shapes=[
            pltpu.VMEM((BG, HC), jnp.float32),
            pltpu.VMEM((1, BG), jnp.float32),
        ],
    )(acc, batch_f, bias, mlp_W1, mlp_b1, mlp_W2, mlp_b2)


def _perm_within_head():
    perm = []
    for p in range(128):
        j, i = divmod(p, 16)
        k, s = divmod(j, 2)
        perm.append(32 * k + 2 * i + s)
    return perm


def kernel(x, edge_index, batch, W_l, W_r, att, bias,
           mlp_W1, mlp_b1, mlp_W2, mlp_b2):
    E = edge_index.shape[1]
    Et = E + N
    ept = -(-Et // (NTILES * 4 * EB)) * 4 * EB
    Et_pad = ept * NTILES
    nb = ept // EB

    loop = jnp.arange(N, dtype=jnp.int32)
    pad = jnp.full((Et_pad - Et,), N, dtype=jnp.int32)
    src = jnp.concatenate([edge_index[0].astype(jnp.int32), loop, pad])
    dst = jnp.concatenate([edge_index[1].astype(jnp.int32), loop, pad])

    hoffs = (jnp.arange(4, dtype=jnp.int32) * N_PAD)[:, None]
    srcoff = (hoffs + src[None, :]).reshape(-1)
    dstoff = (hoffs + dst[None, :]).reshape(-1)

    x_pad = jnp.zeros((N_PAD, D_IN), jnp.float32).at[:N].set(x)
    TL, TR = _prep(x_pad, W_l, W_r)
    TLp = _pack_bf16(TL.reshape(H * N_PAD, DLF))
    TRp = _pack_bf16(TR.reshape(H * N_PAD, C))

    perm = _perm_within_head()
    att_perm = att[:, jnp.array(perm, dtype=jnp.int32)].reshape(H * 8, 16)

    acc = _sc_edge_kernel(ept, nb)(TLp, TRp, srcoff, dstoff, dst, att_perm)
    acc = acc.reshape(H, N_PAD, DM)

    perm_ix = jnp.array(perm, dtype=jnp.int32)
    w1_perm = mlp_W1.reshape(H, C, C)[:, perm_ix, :].reshape(HC, C)
    bias_perm = bias.reshape(H, C)[:, perm_ix].reshape(1, HC)

    batch_f = jnp.full((N_PAD, 1), -1.0, jnp.float32).at[:N, 0].set(
        batch.astype(jnp.float32))
    return _post(acc, batch_f, bias_perm, w1_perm,
                 mlp_b1.reshape(1, C), mlp_W2, mlp_b2.reshape(1, EMBED))


# speedup vs baseline: 1.6922x; 1.6007x over previous
import functools

import jax
import jax.numpy as jnp
from jax import lax
from jax.experimental import pallas as pl
from jax.experimental.pallas import tpu as pltpu
from jax.experimental.pallas import tpu_sc as plsc

N = 10000
D_IN = 128
H = 4
C = 128
HC = H * C
BG = 64
EMBED = 10

N_PAD = 10112
DLF = 160
DLW = DLF // 2
DRW = C // 2
DM = 144
NTILES = 16
EB = 64
ROWS_PER_TILE = N_PAD // NTILES
NBLK_PREP = N_PAD // 128


def _prep_body(x_ref, wl_ref, wr_ref, tl_ref, tr_ref):
    xb = x_ref[...]
    yl = jnp.dot(xb, wl_ref[...], preferred_element_type=jnp.float32)
    yr = jnp.dot(xb, wr_ref[...], preferred_element_type=jnp.float32)
    ones_col = jnp.where(
        lax.broadcasted_iota(jnp.int32, (128, 32), 1) == 0, 1.0, 0.0
    ).astype(jnp.float32)
    for h in range(H):
        tl_ref[h, :, 0:128] = yl[:, h * 128:(h + 1) * 128]
        tl_ref[h, :, 128:160] = ones_col
        tr_ref[h, :, :] = yr[:, h * 128:(h + 1) * 128]


def _prep(x_pad, W_l, W_r):
    return pl.pallas_call(
        _prep_body,
        grid=(NBLK_PREP,),
        in_specs=[
            pl.BlockSpec((128, D_IN), lambda i: (i, 0)),
            pl.BlockSpec((D_IN, HC), lambda i: (0, 0)),
            pl.BlockSpec((D_IN, HC), lambda i: (0, 0)),
        ],
        out_specs=[
            pl.BlockSpec((H, 128, DLF), lambda i: (0, i, 0)),
            pl.BlockSpec((H, 128, C), lambda i: (0, i, 0)),
        ],
        out_shape=[
            jax.ShapeDtypeStruct((H, N_PAD, DLF), jnp.float32),
            jax.ShapeDtypeStruct((H, N_PAD, C), jnp.float32),
        ],
    )(x_pad, W_l, W_r)


def _pack_bf16(arr2d):
    b = arr2d.astype(jnp.bfloat16)
    return lax.bitcast_convert_type(
        b.reshape(arr2d.shape[0], arr2d.shape[1] // 2, 2), jnp.int32)


def _sc_edge_kernel(ept, nb):
    mesh = plsc.VectorSubcoreMesh(
        core_axis_name="c", subcore_axis_name="s", num_cores=2,
        num_subcores=NTILES)

    @functools.partial(
        pl.kernel,
        mesh=mesh,
        compiler_params=pltpu.CompilerParams(
            needs_layout_passes=False, use_tc_tiling_on_sc=False),
        out_type=jax.ShapeDtypeStruct((H * N_PAD, DM), jnp.float32),
        scratch_types=[
            pltpu.VMEM_SHARED((N_PAD, DM), jnp.float32),
            pltpu.VMEM((EB, DLW), jnp.int32),
            pltpu.VMEM((EB, DLW), jnp.int32),
            pltpu.VMEM((EB, DRW), jnp.int32),
            pltpu.VMEM((EB, DRW), jnp.int32),
            pltpu.VMEM((EB, DM), jnp.float32),
            pltpu.VMEM((EB, DM), jnp.float32),
            pltpu.VMEM((EB,), jnp.int32),
            pltpu.VMEM((EB,), jnp.int32),
            pltpu.VMEM((EB,), jnp.int32),
            pltpu.VMEM((EB,), jnp.int32),
            pltpu.VMEM((EB,), jnp.int32),
            pltpu.VMEM((EB,), jnp.int32),
            pltpu.VMEM((EB,), jnp.int32),
            pltpu.VMEM((EB,), jnp.int32),
            pltpu.VMEM((8, 16), jnp.float32),
            pltpu.SemaphoreType.DMA,
            pltpu.SemaphoreType.DMA,
            pltpu.SemaphoreType.DMA,
            pltpu.SemaphoreType.DMA,
            pltpu.SemaphoreType.DMA,
            pltpu.SemaphoreType.DMA,
            pltpu.SemaphoreType.DMA,
            pltpu.SemaphoreType.DMA,
            pltpu.SemaphoreType.DMA,
            pltpu.SemaphoreType.DMA,
            pltpu.SemaphoreType.DMA,
            pltpu.SemaphoreType.DMA,
        ],
    )
    def sc_kernel(tl_hbm, tr_hbm, srcoff_hbm, dstoff_hbm, draw_hbm, att_hbm,
                  out_hbm,
                  accum, l0, l1, r0, r1, m0, m1,
                  so0, so1, do0, do1, db0, db1, db2, db3, attb,
                  sl0, sl1, sr0, sr1, si0, si1, ss0, ss1,
                  sd0, sd1, sd2, sd3):
        cid = lax.axis_index("c")
        sid = lax.axis_index("s")
        lbuf = (l0, l1)
        rbuf = (r0, r1)
        mbuf = (m0, m1)
        sobuf = (so0, so1)
        dobuf = (do0, do1)
        dbuf = (db0, db1, db2, db3)
        sem_l = (sl0, sl1)
        sem_r = (sr0, sr1)
        sem_i = (si0, si1)
        sem_s = (ss0, ss1)
        sem_d = (sd0, sd1, sd2, sd3)
        rbase = sid * ROWS_PER_TILE

        def issue_scatter(b, q):
            pltpu.async_copy(mbuf[b], accum.at[dbuf[q]], sem_s[b], add=True)

        def wait_scatter(b, q):
            pltpu.make_async_copy(mbuf[b], accum.at[dbuf[q]],
                                  sem_s[b]).wait()

        def issue_idx_gather(rc, blk, b):
            base = ((rc * NTILES + sid) * nb + blk) * EB
            pltpu.async_copy(srcoff_hbm.at[pl.ds(base, EB)], sobuf[b],
                             sem_i[b])
            pltpu.async_copy(dstoff_hbm.at[pl.ds(base, EB)], dobuf[b],
                             sem_i[b])

        def issue_idx_draw(blk, q):
            dbase = (sid * nb + blk) * EB
            pltpu.async_copy(draw_hbm.at[pl.ds(dbase, EB)], dbuf[q],
                             sem_d[q])

        def wait_idx(b):
            pltpu.make_async_copy(srcoff_hbm.at[pl.ds(0, EB)], sobuf[b],
                                  sem_i[b]).wait()
            pltpu.make_async_copy(dstoff_hbm.at[pl.ds(0, EB)], dobuf[b],
                                  sem_i[b]).wait()

        def wait_draw(q):
            pltpu.make_async_copy(draw_hbm.at[pl.ds(0, EB)], dbuf[q],
                                  sem_d[q]).wait()

        def issue_gathers(b):
            pltpu.async_copy(tl_hbm.at[sobuf[b]], lbuf[b], sem_l[b])
            pltpu.async_copy(tr_hbm.at[dobuf[b]], rbuf[b], sem_r[b])

        def wait_gathers(b):
            pltpu.make_async_copy(tl_hbm.at[sobuf[b]], lbuf[b],
                                  sem_l[b]).wait()
            pltpu.make_async_copy(tr_hbm.at[dobuf[b]], rbuf[b],
                                  sem_r[b]).wait()

        himask = jnp.int32(-65536)

        def unpack2(w):
            lo = plsc.bitcast(jnp.left_shift(w, 16), jnp.float32)
            hi = plsc.bitcast(jnp.bitwise_and(w, himask), jnp.float32)
            return lo, hi

        def compute_block(b):
            avecs = [attb[k, :] for k in range(8)]

            @plsc.parallel_loop(0, EB)
            def ebody(be):
                lw = [lbuf[b][be, pl.ds(16 * k, 16)] for k in range(5)]
                rw = [rbuf[b][be, pl.ds(16 * k, 16)] for k in range(4)]
                fl = []
                for k in range(4):
                    lo, hi = unpack2(lw[k])
                    fl += [lo, hi]
                fr = []
                for k in range(4):
                    lo, hi = unpack2(rw[k])
                    fr += [lo, hi]
                acc = None
                for j in range(8):
                    t = fl[j] + fr[j]
                    y = jnp.maximum(t, 0.2 * t)
                    p = y * avecs[j]
                    acc = p if acc is None else acc + p
                alpha = jnp.sum(acc)
                exv = jnp.exp(jnp.broadcast_to(alpha, (16,)))
                for j in range(8):
                    mbuf[b][be, pl.ds(16 * j, 16)] = fl[j] * exv
                dlo, dhi = unpack2(lw[4])
                mbuf[b][be, pl.ds(128, 16)] = (dlo + dhi) * exv

        zvec = jnp.zeros((16,), jnp.float32)

        for rnd in range(2):
            h = 2 * rnd + cid
            rc = 2 * rnd + cid

            def zbody(i, carry):
                for k in range(DM // 16):
                    m0[i, pl.ds(16 * k, 16)] = zvec
                return carry
            lax.fori_loop(0, EB, zbody, 0)
            for k in range(ROWS_PER_TILE // EB):
                pltpu.sync_copy(m0.at[pl.ds(0, EB)],
                                accum.at[pl.ds(rbase + k * EB, EB)])
            rem = ROWS_PER_TILE % EB
            if rem:
                pltpu.sync_copy(
                    m0.at[pl.ds(0, rem)],
                    accum.at[pl.ds(rbase + (ROWS_PER_TILE // EB) * EB, rem)])
            pltpu.sync_copy(att_hbm.at[pl.ds(h * 8, 8)], attb)
            plsc.subcore_barrier()

            issue_idx_gather(rc, 0, 0)
            issue_idx_gather(rc, 1, 1)
            issue_idx_draw(0, 0)
            issue_idx_draw(1, 1)
            wait_idx(0)
            issue_gathers(0)

            def gbody(g, carry):
                not_last = g < nb // 4 - 1
                for u in range(4):
                    b = u & 1
                    q2 = (u + 2) % 4
                    wait_gathers(b)

                    def next_gathers(g=g, b=b, u=u):
                        wait_idx(1 - b)
                        issue_gathers(1 - b)

                    if u < 3:
                        next_gathers()
                    else:
                        pl.when(not_last)(next_gathers)

                    def prefetch(g=g, b=b, u=u, q2=q2):
                        issue_idx_gather(rc, 4 * g + u + 2, b)
                        issue_idx_draw(4 * g + u + 2, q2)

                    if u < 2:
                        pl.when(g >= 1)(lambda b=b, q2=q2: wait_scatter(b, q2))
                        prefetch()
                    else:
                        wait_scatter(b, q2)
                        pl.when(not_last)(prefetch)
                    compute_block(b)
                    wait_draw(u)
                    issue_scatter(b, u)
                return carry

            lax.fori_loop(0, nb // 4, gbody, 0)
            wait_scatter(0, (nb - 2) % 4)
            wait_scatter(1, (nb - 1) % 4)
            plsc.subcore_barrier()
            pltpu.sync_copy(
                accum.at[pl.ds(rbase, ROWS_PER_TILE)],
                out_hbm.at[pl.ds(h * N_PAD + rbase, ROWS_PER_TILE)])

    return sc_kernel


POST_R = 1264
POST_G = N_PAD // POST_R


def _post_body(acc_ref, batch_ref, bias_ref, w1_ref, b1_ref, w2_ref, b2_ref,
               out_ref, pooled_s, cnt_s):
    i = pl.program_id(0)

    @pl.when(i == 0)
    def _():
        pooled_s[...] = jnp.zeros_like(pooled_s)
        cnt_s[...] = jnp.zeros_like(cnt_s)

    b = batch_ref[...]
    gid = lax.broadcasted_iota(jnp.int32, (1, BG), 1).astype(jnp.float32)
    P = (b == gid).astype(jnp.float32)
    parts = []
    for h in range(H):
        den = jnp.sum(acc_ref[h, :, 128:144], axis=1, keepdims=True)
        parts.append(acc_ref[h, :, 0:128] / jnp.maximum(den, 1e-30))
    nodes = jnp.concatenate(parts, axis=1)
    pooled_s[...] += lax.dot_general(
        P, nodes, (((0,), (0,)), ((), ())), preferred_element_type=jnp.float32)
    cnt_s[...] += jnp.sum(P, axis=0, keepdims=True)

    @pl.when(i == POST_G - 1)
    def _():
        cnt = jnp.maximum(cnt_s[...], 1.0)
        pm = pooled_s[...] / cnt.reshape(BG, 1) + bias_ref[...]
        hmid = jnp.maximum(
            jnp.dot(pm, w1_ref[...], preferred_element_type=jnp.float32)
            + b1_ref[...], 0.0)
        out_ref[...] = (
            jnp.dot(hmid, w2_ref[...], preferred_element_type=jnp.float32)
            + b2_ref[...])


def _post(acc, batch_f, bias, mlp_W1, mlp_b1, mlp_W2, mlp_b2):
    return pl.pallas_call(
        _post_body,
        grid=(POST_G,),
        in_specs=[
            pl.BlockSpec((H, POST_R, DM), lambda i: (0, i, 0)),
            pl.BlockSpec((POST_R, 1), lambda i: (i, 0)),
            pl.BlockSpec((1, HC), lambda i: (0, 0)),
            pl.BlockSpec((HC, C), lambda i: (0, 0)),
            pl.BlockSpec((1, C), lambda i: (0, 0)),
            pl.BlockSpec((C, EMBED), lambda i: (0, 0)),
            pl.BlockSpec((1, EMBED), lambda i: (0, 0)),
        ],
        out_specs=pl.BlockSpec((BG, EMBED), lambda i: (0, 0)),
        out_shape=jax.ShapeDtypeStruct((BG, EMBED), jnp.float32),
        scratch_shapes=[
            pltpu.VMEM((BG, HC), jnp.float32),
            pltpu.VMEM((1, BG), jnp.float32),
        ],
    )(acc, batch_f, bias, mlp_W1, mlp_b1, mlp_W2, mlp_b2)


def _perm_within_head():
    perm = []
    for p in range(128):
        j, i = divmod(p, 16)
        k, s = divmod(j, 2)
        perm.append(32 * k + 2 * i + s)
    return perm


def kernel(x, edge_index, batch, W_l, W_r, att, bias,
           mlp_W1, mlp_b1, mlp_W2, mlp_b2):
    E = edge_index.shape[1]
    Et = E + N
    ept = -(-Et // (NTILES * 4 * EB)) * 4 * EB
    Et_pad = ept * NTILES
    nb = ept // EB

    loop = jnp.arange(N, dtype=jnp.int32)
    pad = jnp.full((Et_pad - Et,), N, dtype=jnp.int32)
    src = jnp.concatenate([edge_index[0].astype(jnp.int32), loop, pad])
    dst = jnp.concatenate([edge_index[1].astype(jnp.int32), loop, pad])

    hoffs = (jnp.arange(4, dtype=jnp.int32) * N_PAD)[:, None]
    srcoff = (hoffs + src[None, :]).reshape(-1)
    dstoff = (hoffs + dst[None, :]).reshape(-1)

    x_pad = jnp.zeros((N_PAD, D_IN), jnp.float32).at[:N].set(x)
    TL, TR = _prep(x_pad, W_l, W_r)
    TLp = _pack_bf16(TL.reshape(H * N_PAD, DLF))
    TRp = _pack_bf16(TR.reshape(H * N_PAD, C))

    perm = _perm_within_head()
    att_perm = att[:, jnp.array(perm, dtype=jnp.int32)].reshape(H * 8, 16)

    acc = _sc_edge_kernel(ept, nb)(TLp, TRp, srcoff, dstoff, dst, att_perm)
    acc = acc.reshape(H, N_PAD, DM)

    perm_ix = jnp.array(perm, dtype=jnp.int32)
    w1_perm = mlp_W1.reshape(H, C, C)[:, perm_ix, :].reshape(HC, C)
    bias_perm = bias.reshape(H, C)[:, perm_ix].reshape(1, HC)

    batch_f = jnp.full((N_PAD, 1), -1.0, jnp.float32).at[:N, 0].set(
        batch.astype(jnp.float32))
    return _post(acc, batch_f, bias_perm, w1_perm,
                 mlp_b1.reshape(1, C), mlp_W2, mlp_b2.reshape(1, EMBED))
